# baseline (speedup 1.0000x reference)
import functools

import jax
import jax.numpy as jnp
from jax.experimental import pallas as pl
from jax.experimental.pallas import tpu as pltpu

_ANCHORS = ((10.0, 14.0), (23.0, 27.0), (37.0, 58.0),
            (81.0, 82.0), (135.0, 169.0), (344.0, 319.0))
_IGNORE_THRESH = 0.5
_NO_OBJECT_COEFF = 0.5
_COORD_COEFF = 5.0
_SMALL_OFFSET = 507.0


def _softplus(x):
    return jnp.maximum(x, 0.0) + jnp.log1p(jnp.exp(-jnp.abs(x)))


def _bce(x, z):
    return jnp.maximum(x, 0.0) - x * z + jnp.log1p(jnp.exp(-jnp.abs(x)))


def _loss_kernel(nt_ref, pred_ref, tgtT_ref, anch_ref,
                 total_ref, coord_ref, obj_ref, noobj_ref, class_ref,
                 *, B, N, T, A):
    b = pl.program_id(0)

    @pl.when(b == 0)
    def _init():
        total_ref[0, 0] = 0.0
        coord_ref[0, 0] = 0.0
        obj_ref[0, 0] = 0.0
        noobj_ref[0, 0] = 0.0
        class_ref[0, 0] = 0.0

    nt = nt_ref[b]
    tT = tgtT_ref[0]
    txr = tT[0:1, :]
    tyr = tT[1:2, :]
    twr = tT[2:3, :]
    thr = tT[3:4, :]

    lane_t = jax.lax.broadcasted_iota(jnp.int32, (1, T), 1)
    valid = lane_t < jnp.minimum(nt, T)
    validf = valid.astype(jnp.float32)

    wa = anch_ref[:, 0:1]
    ha = anch_ref[:, 1:2]
    wmin = jnp.maximum(jnp.minimum(wa, twr), 0.0)
    hmin = jnp.maximum(jnp.minimum(ha, thr), 0.0)
    inter_a = wmin * hmin
    iou_at = inter_a / (wa * ha + twr * thr - inter_a + 1e-09)
    m = jnp.max(iou_at, axis=0, keepdims=True)
    rows6 = jax.lax.broadcasted_iota(jnp.int32, (6, T), 0)
    aidx = jnp.min(jnp.where(iou_at == m, rows6, 6), axis=0, keepdims=True)

    small = aidx < 3
    stride = jnp.where(small, 16.0, 32.0)
    grid = jnp.where(small, 26.0, 13.0)
    cx = jnp.floor(txr / stride)
    cy = jnp.floor(tyr / stride)
    fx = jnp.clip(txr / stride - cx, 1e-09, 1.0 - 1e-09)
    tx = jnp.log(fx / (1.0 - fx))
    fy = jnp.clip(tyr / stride - cy, 1e-09, 1.0 - 1e-09)
    ty = jnp.log(fy / (1.0 - fy))
    onehot_a = (rows6 == aidx).astype(jnp.float32)
    chosen_w = jnp.sum(onehot_a * wa, axis=0, keepdims=True)
    chosen_h = jnp.sum(onehot_a * ha, axis=0, keepdims=True)
    tw = jnp.log(twr / chosen_w)
    th = jnp.log(thr / chosen_h)
    amod = (aidx - jnp.where(small, 0, 3)).astype(jnp.float32)
    lsm = small.astype(jnp.float32)
    obj_index = (lsm * _SMALL_OFFSET + grid * grid * amod
                 + grid * cy + cx).astype(jnp.int32)

    blk = pred_ref[0]
    px = blk[:, 0:1]
    py = blk[:, 1:2]
    pw = blk[:, 2:3]
    ph = blk[:, 3:4]
    pconf = blk[:, 4:5]

    px1 = px - pw * 0.5
    py1 = py - ph * 0.5
    tx1 = txr - twr * 0.5
    ty1 = tyr - thr * 0.5
    wI = jnp.maximum(jnp.minimum(px1 + pw, tx1 + twr) - jnp.maximum(px1, tx1), 0.0)
    hI = jnp.maximum(jnp.minimum(py1 + ph, ty1 + thr) - jnp.maximum(py1, ty1), 0.0)
    inter = wI * hI
    iou = inter / (pw * ph + twr * thr - inter + 1e-09)
    iou = jnp.where(valid, iou, -1.0)
    maxiou = jnp.max(iou, axis=1, keepdims=True)

    rown = jax.lax.broadcasted_iota(jnp.int32, (N, T), 0)
    onehot = jnp.where(valid, (rown == obj_index).astype(jnp.float32), 0.0)
    assigned = jnp.max(onehot, axis=1, keepdims=True)
    noobj = jnp.logical_and(maxiou < _IGNORE_THRESH, assigned == 0.0)
    noobj_ref[0, 0] += jnp.sum(jnp.where(noobj, _softplus(pconf), 0.0))

    pobjT = jax.lax.dot_general(blk, onehot, (((0,), (0,)), ((), ())),
                                preferred_element_type=jnp.float32)

    rowa = jax.lax.broadcasted_iota(jnp.int32, (A, T), 0)
    tgt_full = jnp.where(rowa == 0, tx,
               jnp.where(rowa == 1, ty,
               jnp.where(rowa == 2, tw,
               jnp.where(rowa == 3, th, tT))))

    vmask = jnp.logical_and(valid, rowa >= 0)
    diff = pobjT - tgt_full
    coord = jnp.sum(jnp.where(jnp.logical_and(rowa < 4, valid), diff * diff, 0.0))
    E = _bce(pobjT, tgt_full)
    objl = jnp.sum(jnp.where(jnp.logical_and(rowa == 4, valid), E, 0.0))
    clsl = jnp.sum(jnp.where(jnp.logical_and(rowa >= 5, valid), E, 0.0))
    del vmask

    coord_ref[0, 0] += coord
    obj_ref[0, 0] += objl
    class_ref[0, 0] += clsl

    @pl.when(b == B - 1)
    def _fin():
        total_ref[0, 0] = (class_ref[0, 0] + obj_ref[0, 0]
                           + _COORD_COEFF * coord_ref[0, 0]
                           + _NO_OBJECT_COEFF * noobj_ref[0, 0])


def kernel(predictions, targets, num_targets):
    B, N, A = predictions.shape
    T = targets.shape[1]
    tgtT = jnp.swapaxes(targets, 1, 2)
    anchors = jnp.asarray(_ANCHORS, dtype=jnp.float32)

    out_shapes = [jax.ShapeDtypeStruct((1, 1), jnp.float32) for _ in range(5)]
    smem_spec = pl.BlockSpec(memory_space=pltpu.SMEM)
    outs = pl.pallas_call(
        functools.partial(_loss_kernel, B=B, N=N, T=T, A=A),
        grid=(B,),
        in_specs=[
            smem_spec,
            pl.BlockSpec((1, N, A), lambda b: (b, 0, 0)),
            pl.BlockSpec((1, A, T), lambda b: (b, 0, 0)),
            pl.BlockSpec((6, 2), lambda b: (0, 0)),
        ],
        out_specs=[smem_spec] * 5,
        out_shape=out_shapes,
    )(num_targets, predictions, tgtT, anchors)
    total, coord, obj, noobj, cls = [o[0, 0] for o in outs]
    return (total, coord, obj, noobj, cls)


# speedup vs baseline: 2.1304x; 2.1304x over previous
import functools

import jax
import jax.numpy as jnp
from jax.experimental import pallas as pl
from jax.experimental.pallas import tpu as pltpu

_ANCHORS_W = (10.0, 23.0, 37.0, 81.0, 135.0, 344.0)
_ANCHORS_H = (14.0, 27.0, 58.0, 82.0, 169.0, 319.0)
_IGNORE_THRESH = 0.5
_NO_OBJECT_COEFF = 0.5
_COORD_COEFF = 5.0
_SMALL_OFFSET = 507.0


def _softplus(x):
    return jnp.maximum(x, 0.0) + jnp.log1p(jnp.exp(-jnp.abs(x)))


def _bce(x, z):
    return jnp.maximum(x, 0.0) - x * z + jnp.log1p(jnp.exp(-jnp.abs(x)))


def _loss_kernel(nt_ref, pred_ref, tgt_ref, anch_ref,
                 total_ref, coord_ref, obj_ref, noobj_ref, class_ref,
                 *, B, N, T, A):
    b = pl.program_id(0)

    @pl.when(b == 0)
    def _init():
        total_ref[0, 0] = 0.0
        coord_ref[0, 0] = 0.0
        obj_ref[0, 0] = 0.0
        noobj_ref[0, 0] = 0.0
        class_ref[0, 0] = 0.0

    nt = nt_ref[b]
    tgt = tgt_ref[0]
    txc = tgt[:, 0:1]
    tyc = tgt[:, 1:2]
    twc = tgt[:, 2:3]
    thc = tgt[:, 3:4]

    sub_t = jax.lax.broadcasted_iota(jnp.int32, (T, 1), 0)
    validb = sub_t < jnp.minimum(nt, T)
    validf = validb.astype(jnp.float32)

    wa = anch_ref[0:1, :]
    ha = anch_ref[1:2, :]
    inter_a = jnp.minimum(wa, twc) * jnp.minimum(ha, thc)
    iou_at = inter_a / (wa * ha + twc * thc - inter_a + 1e-09)
    m = jnp.max(iou_at, axis=1, keepdims=True)
    lane6 = jax.lax.broadcasted_iota(jnp.int32, (T, 6), 1)
    aidx = jnp.min(jnp.where(iou_at == m, lane6, 6), axis=1, keepdims=True)

    small = aidx < 3
    stride = jnp.where(small, 16.0, 32.0)
    grid = jnp.where(small, 26.0, 13.0)
    cx = jnp.floor(txc / stride)
    cy = jnp.floor(tyc / stride)
    fx = jnp.clip(txc / stride - cx, 1e-09, 1.0 - 1e-09)
    tx = jnp.log(fx / (1.0 - fx))
    fy = jnp.clip(tyc / stride - cy, 1e-09, 1.0 - 1e-09)
    ty = jnp.log(fy / (1.0 - fy))
    onehot_a = (lane6 == aidx).astype(jnp.float32)
    chosen_w = jnp.sum(onehot_a * wa, axis=1, keepdims=True)
    chosen_h = jnp.sum(onehot_a * ha, axis=1, keepdims=True)
    tw = jnp.log(twc / chosen_w)
    th = jnp.log(thc / chosen_h)
    amod = (aidx - jnp.where(small, 0, 3)).astype(jnp.float32)
    lsm = small.astype(jnp.float32)
    obj_index = (lsm * _SMALL_OFFSET + grid * grid * amod
                 + grid * cy + cx)

    blk = pred_ref[0]
    blkT = jnp.swapaxes(blk[:, 0:8], 0, 1)
    pxr = blkT[0:1, :]
    pyr = blkT[1:2, :]
    pwr = blkT[2:3, :]
    phr = blkT[3:4, :]
    pconf = blkT[4:5, :]

    px1 = pxr - pwr * 0.5
    py1 = pyr - phr * 0.5
    px2 = px1 + pwr
    py2 = py1 + phr
    aPe = pwr * phr + 1e-09
    tx1 = txc - twc * 0.5
    ty1 = tyc - thc * 0.5
    tx2 = tx1 + twc
    ty2 = ty1 + thc
    aT = twc * thc

    wI = jnp.maximum(jnp.minimum(px2, tx2) - jnp.maximum(px1, tx1), 0.0)
    hI = jnp.maximum(jnp.minimum(py2, ty2) - jnp.maximum(py1, ty1), 0.0)
    inter = wI * hI
    S = aPe + aT
    ge = jnp.logical_and(3.0 * inter >= S, S > inter)

    lane_n = jax.lax.broadcasted_iota(jnp.int32, (T, N), 1)
    eq = lane_n == obj_index.astype(jnp.int32)
    ohTN = jnp.where(jnp.logical_and(eq, validb), 1.0, 0.0)
    covf = jnp.maximum(jnp.where(jnp.logical_and(ge, validb), 1.0, 0.0), ohTN)
    covered = jnp.max(covf, axis=0, keepdims=True)
    noobj_ref[0, 0] += jnp.sum(
        jnp.where(covered == 0.0, _softplus(pconf), 0.0))

    pobj = jax.lax.dot_general(ohTN, blk, (((1,), (0,)), ((), ())),
                               preferred_element_type=jnp.float32)

    lane_a = jax.lax.broadcasted_iota(jnp.int32, (T, A), 1)
    tgt_full = jnp.where(lane_a == 0, tx,
               jnp.where(lane_a == 1, ty,
               jnp.where(lane_a == 2, tw,
               jnp.where(lane_a == 3, th, tgt))))

    diff = pobj - tgt_full
    coord = jnp.sum(jnp.where(jnp.logical_and(lane_a < 4, validb), diff * diff, 0.0))
    E = _bce(pobj, tgt_full)
    objl = jnp.sum(jnp.where(jnp.logical_and(lane_a == 4, validb), E, 0.0))
    clsl = jnp.sum(jnp.where(jnp.logical_and(lane_a >= 5, validb), E, 0.0))

    coord_ref[0, 0] += coord
    obj_ref[0, 0] += objl
    class_ref[0, 0] += clsl

    @pl.when(b == B - 1)
    def _fin():
        total_ref[0, 0] = (class_ref[0, 0] + obj_ref[0, 0]
                           + _COORD_COEFF * coord_ref[0, 0]
                           + _NO_OBJECT_COEFF * noobj_ref[0, 0])


def kernel(predictions, targets, num_targets):
    B, N, A = predictions.shape
    T = targets.shape[1]
    anchors = jnp.asarray([_ANCHORS_W, _ANCHORS_H], dtype=jnp.float32)

    out_shapes = [jax.ShapeDtypeStruct((1, 1), jnp.float32) for _ in range(5)]
    smem_spec = pl.BlockSpec(memory_space=pltpu.SMEM)
    outs = pl.pallas_call(
        functools.partial(_loss_kernel, B=B, N=N, T=T, A=A),
        grid=(B,),
        in_specs=[
            smem_spec,
            pl.BlockSpec((1, N, A), lambda b: (b, 0, 0)),
            pl.BlockSpec((1, T, A), lambda b: (b, 0, 0)),
            pl.BlockSpec((2, 6), lambda b: (0, 0)),
        ],
        out_specs=[smem_spec] * 5,
        out_shape=out_shapes,
    )(num_targets, predictions, targets, anchors)
    total, coord, obj, noobj, cls = [o[0, 0] for o in outs]
    return (total, coord, obj, noobj, cls)
